# baseline (speedup 1.0000x reference)
import functools

import jax
import jax.numpy as jnp
from jax import lax
from jax.experimental import pallas as pl
from jax.experimental.pallas import tpu as pltpu
from jax.experimental.pallas import tpu_sc as plsc

N_OBJS = 100000
EMBED_DIM = 32
B_ROWS = 16384
SEQ = 50
B_TOTAL = B_ROWS * SEQ

_info = plsc.get_sparse_core_info()
NC, NS = _info.num_cores, _info.num_subcores
NW = NC * NS
B_PER_W = B_TOTAL // NW
ROWS_PER_W = B_ROWS // NW
CHUNK_ROWS = 32
CHUNK = CHUNK_ROWS * SEQ
CHUNKS = ROWS_PER_W // CHUNK_ROWS
NBUF = 2

_mesh = plsc.VectorSubcoreMesh(core_axis_name="c", subcore_axis_name="s")


@functools.partial(
    pl.kernel,
    mesh=_mesh,
    out_type=jax.ShapeDtypeStruct((B_ROWS, 56, 128), jnp.float32),
    scratch_types=[
        [pltpu.VMEM((CHUNK,), jnp.int32) for _ in range(NBUF)],
        [pltpu.VMEM((CHUNK, EMBED_DIM), jnp.float32) for _ in range(NBUF)],
        [pltpu.SemaphoreType.DMA for _ in range(NBUF)],
        [pltpu.SemaphoreType.DMA for _ in range(NBUF)],
        [pltpu.SemaphoreType.DMA for _ in range(NBUF)],
    ],
    compiler_params=pltpu.CompilerParams(use_tc_tiling_on_sc=False),
)
def _gather_kernel(table_hbm, idx_hbm, out_hbm, idx_v, rows_v, si, sg, so):
    wid = lax.axis_index("s") * NC + lax.axis_index("c")
    wbase = wid * B_PER_W
    wrow = wid * ROWS_PER_W

    def start_idx(c, b):
        base = wbase + c * CHUNK
        pltpu.async_copy(idx_hbm.at[pl.ds(base, CHUNK)], idx_v[b], si[b])

    def start_out(c, b):
        row0 = wrow + c * CHUNK_ROWS
        for j in range(CHUNK_ROWS):
            pltpu.async_copy(
                rows_v[b].at[pl.ds(j * SEQ, SEQ)],
                out_hbm.at[row0 + j, pl.ds(0, SEQ), pl.ds(0, EMBED_DIM)],
                so[b])

    def wait_out(b):
        for _ in range(CHUNK_ROWS):
            pltpu.make_async_copy(
                rows_v[b].at[pl.ds(0, SEQ)],
                out_hbm.at[0, pl.ds(0, SEQ), pl.ds(0, EMBED_DIM)],
                so[b]).wait()

    start_idx(0, 0)
    start_idx(1, 1)
    pltpu.make_async_copy(idx_hbm.at[pl.ds(0, CHUNK)], idx_v[0], si[0]).wait()
    pltpu.async_copy(table_hbm.at[idx_v[0]], rows_v[0], sg[0])
    for c in range(CHUNKS):
        b = c % NBUF
        nb = (c + 1) % NBUF
        if c + 1 < CHUNKS:
            pltpu.make_async_copy(
                idx_hbm.at[pl.ds(0, CHUNK)], idx_v[nb], si[nb]).wait()
            if c + 1 >= NBUF:
                wait_out(nb)
            pltpu.async_copy(table_hbm.at[idx_v[nb]], rows_v[nb], sg[nb])
        pltpu.make_async_copy(table_hbm.at[idx_v[b]], rows_v[b], sg[b]).wait()
        start_out(c, b)
        if c + NBUF < CHUNKS:
            start_idx(c + NBUF, b)
    for b in range(NBUF):
        wait_out(b)


def kernel(x, table):
    idx = x.reshape(-1).astype(jnp.int32)
    out56 = _gather_kernel(table, idx)
    return out56[:, :SEQ, :EMBED_DIM].reshape(x.shape + (EMBED_DIM,))


# speedup vs baseline: 1.0026x; 1.0026x over previous
import functools

import jax
import jax.numpy as jnp
from jax import lax
from jax.experimental import pallas as pl
from jax.experimental.pallas import tpu as pltpu
from jax.experimental.pallas import tpu_sc as plsc

N_OBJS = 100000
EMBED_DIM = 32
B_ROWS = 16384
SEQ = 50
B_TOTAL = B_ROWS * SEQ

_info = plsc.get_sparse_core_info()
NC, NS = _info.num_cores, _info.num_subcores
NW = NC * NS
B_PER_W = B_TOTAL // NW
ROWS_PER_W = B_ROWS // NW
CHUNK_ROWS = 16
CHUNK = CHUNK_ROWS * SEQ
CHUNKS = ROWS_PER_W // CHUNK_ROWS
NBUF = 4

_mesh = plsc.VectorSubcoreMesh(core_axis_name="c", subcore_axis_name="s")


@functools.partial(
    pl.kernel,
    mesh=_mesh,
    out_type=jax.ShapeDtypeStruct((B_ROWS, 56, 128), jnp.float32),
    scratch_types=[
        [pltpu.VMEM((CHUNK,), jnp.int32) for _ in range(NBUF)],
        [pltpu.VMEM((CHUNK, EMBED_DIM), jnp.float32) for _ in range(NBUF)],
        [pltpu.SemaphoreType.DMA for _ in range(NBUF)],
        [pltpu.SemaphoreType.DMA for _ in range(NBUF)],
        [pltpu.SemaphoreType.DMA for _ in range(NBUF)],
    ],
    compiler_params=pltpu.CompilerParams(use_tc_tiling_on_sc=False),
)
def _gather_kernel(table_hbm, idx_hbm, out_hbm, idx_v, rows_v, si, sg, so):
    wid = lax.axis_index("s") * NC + lax.axis_index("c")
    wbase = wid * B_PER_W
    wrow = wid * ROWS_PER_W

    def start_idx(c, b):
        base = wbase + c * CHUNK
        pltpu.async_copy(idx_hbm.at[pl.ds(base, CHUNK)], idx_v[b], si[b])

    def start_out(c, b):
        row0 = wrow + c * CHUNK_ROWS
        for j in range(CHUNK_ROWS):
            pltpu.async_copy(
                rows_v[b].at[pl.ds(j * SEQ, SEQ)],
                out_hbm.at[row0 + j, pl.ds(0, SEQ), pl.ds(0, EMBED_DIM)],
                so[b])

    def wait_out(b):
        for _ in range(CHUNK_ROWS):
            pltpu.make_async_copy(
                rows_v[b].at[pl.ds(0, SEQ)],
                out_hbm.at[0, pl.ds(0, SEQ), pl.ds(0, EMBED_DIM)],
                so[b]).wait()

    for k in range(min(NBUF, CHUNKS)):
        start_idx(k, k)
    pltpu.make_async_copy(idx_hbm.at[pl.ds(0, CHUNK)], idx_v[0], si[0]).wait()
    pltpu.async_copy(table_hbm.at[idx_v[0]], rows_v[0], sg[0])
    for c in range(CHUNKS):
        b = c % NBUF
        nb = (c + 1) % NBUF
        if c + 1 < CHUNKS:
            pltpu.make_async_copy(
                idx_hbm.at[pl.ds(0, CHUNK)], idx_v[nb], si[nb]).wait()
            if c + 1 >= NBUF:
                wait_out(nb)
            pltpu.async_copy(table_hbm.at[idx_v[nb]], rows_v[nb], sg[nb])
        pltpu.make_async_copy(table_hbm.at[idx_v[b]], rows_v[b], sg[b]).wait()
        start_out(c, b)
        if c + NBUF < CHUNKS:
            start_idx(c + NBUF, b)
    for b in range(NBUF):
        wait_out(b)


def kernel(x, table):
    idx = x.reshape(-1).astype(jnp.int32)
    out56 = _gather_kernel(table, idx)
    return out56[:, :SEQ, :EMBED_DIM].reshape(x.shape + (EMBED_DIM,))
